# baseline (speedup 1.0000x reference)
import functools

import jax
import jax.numpy as jnp
from jax import lax
from jax.experimental import pallas as pl
from jax.experimental.pallas import tpu as pltpu
from jax.experimental.pallas import tpu_sc as plsc

_B, _S, _H = 2, 8192, 4096
_E = 64
_TOPK = 8
_ZC = 0.001
_N = _B * _S

_T = 1024
_GRID = _N // _T

_NW = 32
_TPW = _N // _NW
_L = 16


def _router_body(x_ref, wt_ref, rw_ref, z_ref, lb_ref, zacc_ref, uacc_ref):
    i = pl.program_id(0)

    x = x_ref[...].astype(jnp.bfloat16)
    wt = wt_ref[...].astype(jnp.bfloat16)
    logits = jax.lax.dot_general(
        x, wt, (((1,), (0,)), ((), ())),
        preferred_element_type=jnp.float32)

    m = jnp.max(logits, axis=-1, keepdims=True)
    e = jnp.exp(logits - m)
    s = jnp.sum(e, axis=-1, keepdims=True)
    rw = e / s
    lse = m + jnp.log(s)

    z_part = jnp.sum(lse * lse)
    u_part = jnp.sum(rw, axis=0, keepdims=True)

    @pl.when(i == 0)
    def _init():
        zacc_ref[0, 0] = z_part
        uacc_ref[...] = u_part

    @pl.when(i > 0)
    def _acc():
        zacc_ref[0, 0] += z_part
        uacc_ref[...] += u_part

    rw_ref[...] = rw

    @pl.when(i == _GRID - 1)
    def _fin():
        z_ref[0, 0] = zacc_ref[0, 0] / _N * _ZC
        usage = uacc_ref[...] / _N
        tgt = 1.0 / _E
        lb = jnp.sum(tgt * (jnp.log(tgt) - jnp.log(usage))) * 0.01
        lb_ref[0, 0] = lb


def _tc_stage(x, wt):
    return pl.pallas_call(
        _router_body,
        grid=(_GRID,),
        in_specs=[
            pl.BlockSpec((_T, _H), lambda i: (i, 0)),
            pl.BlockSpec((_H, _E), lambda i: (0, 0)),
        ],
        out_specs=[
            pl.BlockSpec((_T, _E), lambda i: (i, 0)),
            pl.BlockSpec(memory_space=pltpu.SMEM),
            pl.BlockSpec(memory_space=pltpu.SMEM),
        ],
        out_shape=[
            jax.ShapeDtypeStruct((_N, _E), jnp.float32),
            jax.ShapeDtypeStruct((1, 1), jnp.float32),
            jax.ShapeDtypeStruct((1, 1), jnp.float32),
        ],
        scratch_shapes=[
            pltpu.SMEM((1, 1), jnp.float32),
            pltpu.VMEM((1, _E), jnp.float32),
        ],
    )(x, wt)


def _sc_topk_body(rw_hbm, out_hbm, buf, obuf):
    c = lax.axis_index("c")
    s = lax.axis_index("s")
    wid = s * 2 + c
    base = wid * _TPW

    pltpu.sync_copy(rw_hbm.at[pl.ds(base, _TPW), :], buf)

    neg_inf = jnp.float32(-jnp.inf)

    lanes = lax.iota(jnp.int32, _L)
    perms = [lanes ^ d for d in (8, 4, 2, 1)]

    def token_body(t, carry):
        vs = [buf[t, pl.ds(j * _L, _L)] for j in range(_E // _L)]
        work = list(vs)

        def row_max(ws):
            m = jnp.maximum(jnp.maximum(ws[0], ws[1]),
                            jnp.maximum(ws[2], ws[3]))
            for p in perms:
                m = jnp.maximum(m, m[p])
            return m

        cur = row_max(work)
        for _ in range(_TOPK - 1):
            work = [jnp.where(w == cur, neg_inf, w) for w in work]
            cur = row_max(work)

        kept = [jnp.where(v >= cur, v, 0.0) for v in vs]
        tot = kept[0] + kept[1] + kept[2] + kept[3]
        for p in perms:
            tot = tot + tot[p]
        r = 1.0 / tot
        for j in range(_E // _L):
            obuf[t, pl.ds(j * _L, _L)] = kept[j] * r
        return carry

    lax.fori_loop(0, _TPW, token_body, 0)

    pltpu.sync_copy(obuf, out_hbm.at[pl.ds(base, _TPW), :])


_sc_topk = functools.partial(
    pl.kernel,
    out_type=jax.ShapeDtypeStruct((_N, _E), jnp.float32),
    mesh=plsc.VectorSubcoreMesh(core_axis_name="c", subcore_axis_name="s"),
    scratch_types=[
        pltpu.VMEM((_TPW, _E), jnp.float32),
        pltpu.VMEM((_TPW, _E), jnp.float32),
    ],
)(_sc_topk_body)


@functools.partial(jax.jit, static_argnames=())
def kernel(hidden_states, W):
    x = hidden_states.reshape(_N, _H)
    wt = W.T

    rw, z, lb = _tc_stage(x, wt)
    rw_out = _sc_topk(rw)

    return (rw_out.reshape(_B, _S, _E), z[0, 0], lb[0, 0])


# speedup vs baseline: 1.3999x; 1.3999x over previous
import functools

import jax
import jax.numpy as jnp
from jax.experimental import pallas as pl
from jax.experimental.pallas import tpu as pltpu

_B, _S, _H = 2, 8192, 4096
_E = 64
_TOPK = 8
_ZC = 0.001
_N = _B * _S

_T = 1024
_GRID = _N // _T


def _router_body(x_ref, wt_ref, rw_ref, z_ref, lb_ref, zacc_ref, uacc_ref):
    i = pl.program_id(0)

    x = x_ref[...].astype(jnp.bfloat16)
    wt = wt_ref[...].astype(jnp.bfloat16)
    logits = jax.lax.dot_general(
        x, wt, (((1,), (0,)), ((), ())),
        preferred_element_type=jnp.float32)

    work = logits
    cur = jnp.max(work, axis=-1, keepdims=True)
    m = cur
    for _ in range(_TOPK - 1):
        work = jnp.where(work == cur, -jnp.inf, work)
        cur = jnp.max(work, axis=-1, keepdims=True)
    keep = logits >= cur

    e = jnp.exp(logits - m)
    s = jnp.sum(e, axis=-1, keepdims=True)
    rw = e / s
    lse = m + jnp.log(s)

    z_part = jnp.sum(lse * lse)
    u_part = jnp.sum(rw, axis=0, keepdims=True)

    @pl.when(i == 0)
    def _init():
        zacc_ref[0, 0] = z_part
        uacc_ref[...] = u_part

    @pl.when(i > 0)
    def _acc():
        zacc_ref[0, 0] += z_part
        uacc_ref[...] += u_part

    masked = jnp.where(keep, rw, 0.0)
    rw_ref[...] = masked / jnp.sum(masked, axis=-1, keepdims=True)

    @pl.when(i == _GRID - 1)
    def _fin():
        z_ref[0, 0] = zacc_ref[0, 0] / _N * _ZC
        usage = uacc_ref[...] / _N
        tgt = 1.0 / _E
        lb = jnp.sum(tgt * (jnp.log(tgt) - jnp.log(usage))) * 0.01
        lb_ref[0, 0] = lb


@functools.partial(jax.jit, static_argnames=())
def kernel(hidden_states, W):
    x = hidden_states.reshape(_N, _H)
    wt = W.T

    rw, z, lb = pl.pallas_call(
        _router_body,
        grid=(_GRID,),
        in_specs=[
            pl.BlockSpec((_T, _H), lambda i: (i, 0)),
            pl.BlockSpec((_H, _E), lambda i: (0, 0)),
        ],
        out_specs=[
            pl.BlockSpec((_T, _E), lambda i: (i, 0)),
            pl.BlockSpec(memory_space=pltpu.SMEM),
            pl.BlockSpec(memory_space=pltpu.SMEM),
        ],
        out_shape=[
            jax.ShapeDtypeStruct((_N, _E), jnp.float32),
            jax.ShapeDtypeStruct((1, 1), jnp.float32),
            jax.ShapeDtypeStruct((1, 1), jnp.float32),
        ],
        scratch_shapes=[
            pltpu.SMEM((1, 1), jnp.float32),
            pltpu.VMEM((1, _E), jnp.float32),
        ],
    )(x, wt)

    return (rw.reshape(_B, _S, _E), z[0, 0], lb[0, 0])
